# baseline (speedup 1.0000x reference)
import functools

import jax
import jax.numpy as jnp
from jax import lax
from jax.experimental import pallas as pl
from jax.experimental.pallas import tpu as pltpu
from jax.experimental.pallas import tpu_sc as plsc

N = 50000
E = 800000
D = 64
HD = 32
NG = 16
NUM_LAYERS = 3

NTILES = 16
N_PAD = 50176
NPT = N_PAD // NTILES
EPT = E // NTILES
CH = 80
NCHUNK = EPT // CH

BN = 2000
NBN = N // BN
BE = 8000
NBE = E // BE


def _ln_rows(t, g, b):
    mu = jnp.mean(t, axis=-1, keepdims=True)
    var = jnp.mean((t - mu) * (t - mu), axis=-1, keepdims=True)
    return (t - mu) * lax.rsqrt(var + 1e-5) * g + b


def _prologue_body(x_ref, wn_ref, bn_ref, lng_ref, lnb_ref, hl_ref):
    h = jnp.dot(x_ref[...], wn_ref[...], preferred_element_type=jnp.float32)
    h = jnp.maximum(h + bn_ref[...], 0.0)
    hl_ref[...] = _ln_rows(h, lng_ref[...], lnb_ref[...])


def _node_prologue(x, Wn, bn, ln_g, ln_b):
    return pl.pallas_call(
        _prologue_body,
        grid=(NBN,),
        in_specs=[
            pl.BlockSpec((BN, 153), lambda i: (i, 0)),
            pl.BlockSpec((153, D), lambda i: (0, 0)),
            pl.BlockSpec((1, D), lambda i: (0, 0)),
            pl.BlockSpec((1, D), lambda i: (0, 0)),
            pl.BlockSpec((1, D), lambda i: (0, 0)),
        ],
        out_specs=pl.BlockSpec((BN, D), lambda i: (i, 0)),
        out_shape=jax.ShapeDtypeStruct((N, D), jnp.float32),
    )(x, Wn, bn.reshape(1, D), ln_g.reshape(1, D), ln_b.reshape(1, D))


def _edge_prologue_body(ea_ref, wee_ref, bee_ref, wae_ref, bae_ref,
                        wi1_ref, wi2_ref, wo1_ref, wo2_ref,
                        env_ref, pin_ref, pout_ref):
    ea = ea_ref[...]
    env = jnp.maximum(jnp.dot(ea, wee_ref[...], preferred_element_type=jnp.float32) + bee_ref[...], 0.0)
    act = jnp.maximum(jnp.dot(ea, wae_ref[...], preferred_element_type=jnp.float32) + bae_ref[...], 0.0)
    env_ref[...] = env
    pin_ref[...] = (jnp.dot(env, wi1_ref[...], preferred_element_type=jnp.float32)
                    + jnp.dot(act, wi2_ref[...], preferred_element_type=jnp.float32))
    pout_ref[...] = (jnp.dot(env, wo1_ref[...], preferred_element_type=jnp.float32)
                     + jnp.dot(act, wo2_ref[...], preferred_element_type=jnp.float32))


def _edge_prologue(edge_attr, Wee, bee, Wae, bae, iW1, iW2, oW1, oW2):
    wspec = pl.BlockSpec((7, D), lambda i: (0, 0))
    dspec = pl.BlockSpec((D, D), lambda i: (0, 0))
    bspec = pl.BlockSpec((1, D), lambda i: (0, 0))
    espec = pl.BlockSpec((BE, D), lambda i: (i, 0))
    return pl.pallas_call(
        _edge_prologue_body,
        grid=(NBE,),
        in_specs=[pl.BlockSpec((BE, 7), lambda i: (i, 0)),
                  wspec, bspec, wspec, bspec, dspec, dspec, dspec, dspec],
        out_specs=[espec, espec, espec],
        out_shape=[jax.ShapeDtypeStruct((E, D), jnp.float32)] * 3,
    )(edge_attr, Wee, bee.reshape(1, D), Wae, bae.reshape(1, D), iW1, iW2, oW1, oW2)


def _premul_body(hl_ref, wia_ref, woa_ref, hin_ref, hout_ref):
    hl = hl_ref[...]
    hin_ref[...] = jnp.dot(hl, wia_ref[...], preferred_element_type=jnp.float32)
    hout_ref[...] = jnp.dot(hl, woa_ref[...], preferred_element_type=jnp.float32)


def _premul(hL, in_Wa, out_Wa):
    nspec = pl.BlockSpec((BN, D), lambda i: (i, 0))
    dspec = pl.BlockSpec((D, D), lambda i: (0, 0))
    return pl.pallas_call(
        _premul_body,
        grid=(NBN,),
        in_specs=[nspec, dspec, dspec],
        out_specs=[nspec, nspec],
        out_shape=[jax.ShapeDtypeStruct((N, D), jnp.float32)] * 2,
    )(hL, in_Wa, out_Wa)


def _gates_body(sin_ref, sout_ref, hl_ref,
                wi0_ref, wi1_ref, wo0_ref, wo1_ref,
                gi0_ref, gi1_ref, go0_ref, go1_ref,
                bin_ref, bout_ref, hb_ref):
    a_in = jnp.maximum(sin_ref[...], 0.0)
    a_out = jnp.maximum(sout_ref[...], 0.0)
    li0 = jnp.dot(a_in, wi0_ref[...], preferred_element_type=jnp.float32) + gi0_ref[...]
    li1 = jnp.dot(a_in, wi1_ref[...], preferred_element_type=jnp.float32) + gi1_ref[...]
    lo0 = jnp.dot(a_out, wo0_ref[...], preferred_element_type=jnp.float32) + go0_ref[...]
    lo1 = jnp.dot(a_out, wo1_ref[...], preferred_element_type=jnp.float32) + go1_ref[...]
    b_in = (li0 >= li1).astype(jnp.float32)
    b_out = (lo0 >= lo1).astype(jnp.float32)
    bin_ref[...] = b_in
    bout_ref[...] = b_out
    hb_ref[...] = b_out * hl_ref[...]


def _gates(S_in, S_out, hL, wi0, wi1, wo0, wo1, gi0, gi1, go0, go1):
    nspec = pl.BlockSpec((BN, D), lambda i: (i, 0))
    cspec = pl.BlockSpec((D, 1), lambda i: (0, 0))
    vspec = pl.BlockSpec((BN, 1), lambda i: (i, 0))
    return pl.pallas_call(
        _gates_body,
        grid=(NBN,),
        in_specs=[nspec, nspec, nspec, cspec, cspec, cspec, cspec,
                  vspec, vspec, vspec, vspec],
        out_specs=[vspec, vspec, nspec],
        out_shape=[jax.ShapeDtypeStruct((N, 1), jnp.float32),
                   jax.ShapeDtypeStruct((N, 1), jnp.float32),
                   jax.ShapeDtypeStruct((N, D), jnp.float32)],
    )(S_in, S_out, hL, wi0, wi1, wo0, wo1, gi0, gi1, go0, go1)


def _update_body(hl_ref, t_ref, bin_ref, ws_ref, wm_ref, eb_ref,
                 lng_ref, lnb_ref, out_ref):
    hl = hl_ref[...]
    agg = bin_ref[...] * t_ref[...]
    o = (jnp.dot(hl, ws_ref[...], preferred_element_type=jnp.float32)
         + jnp.dot(agg, wm_ref[...], preferred_element_type=jnp.float32)
         + eb_ref[...])
    r = hl + jnp.maximum(o, 0.0)
    out_ref[...] = _ln_rows(r, lng_ref[...], lnb_ref[...])


def _update(hL, T, b_in, Ws, Wm, Ebi, ln_g, ln_b):
    nspec = pl.BlockSpec((BN, D), lambda i: (i, 0))
    dspec = pl.BlockSpec((D, D), lambda i: (0, 0))
    bspec = pl.BlockSpec((1, D), lambda i: (0, 0))
    return pl.pallas_call(
        _update_body,
        grid=(NBN,),
        in_specs=[nspec, nspec, pl.BlockSpec((BN, 1), lambda i: (i, 0)),
                  dspec, dspec, bspec, bspec, bspec],
        out_specs=nspec,
        out_shape=jax.ShapeDtypeStruct((N, D), jnp.float32),
    )(hL, T, b_in, Ws, Wm, Ebi.reshape(1, D), ln_g.reshape(1, D), ln_b.reshape(1, D))


def _final_body(hl_ref, wf_ref, bf_ref, g1_ref, g1b_ref, g2_ref, g2b_ref,
                hf_ref, gate_ref):
    hf = jnp.dot(hl_ref[...], wf_ref[...], preferred_element_type=jnp.float32) + bf_ref[...]
    hf_ref[...] = hf
    t = jnp.maximum(jnp.dot(hf, g1_ref[...], preferred_element_type=jnp.float32) + g1b_ref[...], 0.0)
    gate_ref[...] = jnp.dot(t, g2_ref[...], preferred_element_type=jnp.float32) + g2b_ref[...]


def _final(hL, Wfin, bfin, G1, g1b, G2, g2b):
    nspec = pl.BlockSpec((BN, D), lambda i: (i, 0))
    dspec = pl.BlockSpec((D, D), lambda i: (0, 0))
    bspec = pl.BlockSpec((1, D), lambda i: (0, 0))
    return pl.pallas_call(
        _final_body,
        grid=(NBN,),
        in_specs=[nspec, dspec, bspec, dspec, bspec,
                  pl.BlockSpec((D, 1), lambda i: (0, 0)),
                  pl.BlockSpec((1, 1), lambda i: (0, 0))],
        out_specs=[nspec, pl.BlockSpec((BN, 1), lambda i: (i, 0))],
        out_shape=[jax.ShapeDtypeStruct((N, D), jnp.float32),
                   jax.ShapeDtypeStruct((N, 1), jnp.float32)],
    )(hL, Wfin, bfin.reshape(1, D), G1, g1b.reshape(1, D), G2,
      g2b.reshape(1, 1))


def _pool_body(gate_ref, hf_ref, oh_ref, out_ref, smax, sden, snum):
    p = pl.program_id(0)
    j = pl.program_id(1)
    oh = oh_ref[...]
    gate = gate_ref[...]

    @pl.when(jnp.logical_and(p == 0, j == 0))
    def _():
        smax[...] = jnp.full((8, NG), -jnp.inf, jnp.float32)

    @pl.when(p == 0)
    def _():
        m = jnp.max(jnp.where(oh > 0.0, gate, -jnp.inf), axis=0, keepdims=True)
        smax[0:1, :] = jnp.maximum(smax[0:1, :], m)

    @pl.when(jnp.logical_and(p == 1, j == 0))
    def _():
        sden[...] = jnp.zeros((8, NG), jnp.float32)
        snum[...] = jnp.zeros((NG, D), jnp.float32)

    @pl.when(p == 1)
    def _():
        gmax_node = jnp.sum(oh * smax[0:1, :], axis=1, keepdims=True)
        ex = jnp.exp(gate - gmax_node)
        sden[0:1, :] = sden[0:1, :] + jnp.sum(oh * ex, axis=0, keepdims=True)
        snum[...] = snum[...] + lax.dot_general(
            oh, ex * hf_ref[...], (((0,), (0,)), ((), ())),
            preferred_element_type=jnp.float32)

    @pl.when(jnp.logical_and(p == 1, j == NBN - 1))
    def _():
        den_col = jnp.reshape(sden[0:1, :], (NG, 1))
        out_ref[...] = snum[...] / (den_col + 1e-16)


def _pool(gate, hF, oh):
    return pl.pallas_call(
        _pool_body,
        grid=(2, NBN),
        in_specs=[pl.BlockSpec((BN, 1), lambda p, j: (j, 0)),
                  pl.BlockSpec((BN, D), lambda p, j: (j, 0)),
                  pl.BlockSpec((BN, NG), lambda p, j: (j, 0))],
        out_specs=pl.BlockSpec((NG, D), lambda p, j: (0, 0)),
        out_shape=jax.ShapeDtypeStruct((NG, D), jnp.float32),
        scratch_shapes=[pltpu.VMEM((8, NG), jnp.float32),
                        pltpu.VMEM((8, NG), jnp.float32),
                        pltpu.VMEM((NG, D), jnp.float32)],
    )(gate, hF, oh)


_MESH = plsc.VectorSubcoreMesh(core_axis_name="c", subcore_axis_name="s")
_SC_PARAMS = pltpu.CompilerParams(use_tc_tiling_on_sc=False,
                                  needs_layout_passes=False)


def _sweep_gather_add_kernel(u2_hbm, v_hbm, table_hbm, env2_hbm,
                             zeros_hbm, out_hbm,
                             uidx_v, vidx_v, rows_v, env_v, sem, shared):
    c = lax.axis_index("c")
    s = lax.axis_index("s")
    pltpu.sync_copy(zeros_hbm.at[pl.ds(s * NPT, NPT)],
                    shared.at[pl.ds(s * NPT, NPT)])
    plsc.subcore_barrier()

    def chunk(i, carry):
        base = s * EPT + i * CH
        pltpu.sync_copy(u2_hbm.at[pl.ds(c * E + base, CH)], uidx_v)
        pltpu.sync_copy(v_hbm.at[pl.ds(base, CH)], vidx_v)
        pltpu.sync_copy(env2_hbm.at[pl.ds(c * E + base, CH)], env_v)
        pltpu.async_copy(table_hbm.at[uidx_v], rows_v, sem).wait()

        def edge(e, carry2):
            rows_v[e, pl.ds(0, 16)] = rows_v[e, pl.ds(0, 16)] + env_v[e, pl.ds(0, 16)]
            rows_v[e, pl.ds(16, 16)] = rows_v[e, pl.ds(16, 16)] + env_v[e, pl.ds(16, 16)]
            return carry2

        lax.fori_loop(0, CH, edge, 0)
        pltpu.sync_copy(rows_v, shared.at[vidx_v], add=True)
        return carry

    lax.fori_loop(0, NCHUNK, chunk, 0)
    plsc.subcore_barrier()
    pltpu.sync_copy(shared.at[pl.ds(s * NPT, NPT)],
                    out_hbm.at[pl.ds(c * N_PAD + s * NPT, NPT)])


def _sweep_gather_env_kernel(u2_hbm, v_hbm, table_hbm, b2_hbm, env2_hbm,
                             zeros_hbm, out_hbm,
                             uidx_v, vidx_v, rows_v, env_v, bval_v, sem, semb,
                             shared):
    c = lax.axis_index("c")
    s = lax.axis_index("s")
    pltpu.sync_copy(zeros_hbm.at[pl.ds(s * NPT, NPT)],
                    shared.at[pl.ds(s * NPT, NPT)])
    plsc.subcore_barrier()

    def chunk(i, carry):
        base = s * EPT + i * CH
        pltpu.sync_copy(u2_hbm.at[pl.ds(c * E + base, CH)], uidx_v)
        pltpu.sync_copy(v_hbm.at[pl.ds(base, CH)], vidx_v)
        pltpu.sync_copy(env2_hbm.at[pl.ds(c * E + base, CH)], env_v)
        cp1 = pltpu.async_copy(table_hbm.at[uidx_v], rows_v, sem)
        cp2 = pltpu.async_copy(b2_hbm.at[uidx_v], bval_v, semb)
        cp1.wait()
        cp2.wait()

        def edge(e, carry2):
            b = plsc.load_gather(bval_v, [jnp.full((16,), e, jnp.int32)])
            lo = rows_v[e, pl.ds(0, 16)]
            hi = rows_v[e, pl.ds(16, 16)]
            elo = env_v[e, pl.ds(0, 16)]
            ehi = env_v[e, pl.ds(16, 16)]
            rows_v[e, pl.ds(0, 16)] = lo + b * elo
            rows_v[e, pl.ds(16, 16)] = hi + b * ehi
            return carry2

        lax.fori_loop(0, CH, edge, 0)
        pltpu.sync_copy(rows_v, shared.at[vidx_v], add=True)
        return carry

    lax.fori_loop(0, NCHUNK, chunk, 0)
    plsc.subcore_barrier()
    pltpu.sync_copy(shared.at[pl.ds(s * NPT, NPT)],
                    out_hbm.at[pl.ds(c * N_PAD + s * NPT, NPT)])


_sweep_gather_add = functools.partial(
    pl.kernel, _sweep_gather_add_kernel, mesh=_MESH, compiler_params=_SC_PARAMS,
    out_type=jax.ShapeDtypeStruct((2 * N_PAD, HD), jnp.float32),
    scratch_types=[pltpu.VMEM((CH,), jnp.int32),
                   pltpu.VMEM((CH,), jnp.int32),
                   pltpu.VMEM((CH, HD), jnp.float32),
                   pltpu.VMEM((CH, HD), jnp.float32),
                   pltpu.SemaphoreType.DMA,
                   pltpu.VMEM_SHARED((N_PAD, HD), jnp.float32)],
)

_sweep_gather_env = functools.partial(
    pl.kernel, _sweep_gather_env_kernel, mesh=_MESH, compiler_params=_SC_PARAMS,
    out_type=jax.ShapeDtypeStruct((2 * N_PAD, HD), jnp.float32),
    scratch_types=[pltpu.VMEM((CH,), jnp.int32),
                   pltpu.VMEM((CH,), jnp.int32),
                   pltpu.VMEM((CH, HD), jnp.float32),
                   pltpu.VMEM((CH, HD), jnp.float32),
                   pltpu.VMEM((CH,), jnp.float32),
                   pltpu.SemaphoreType.DMA,
                   pltpu.SemaphoreType.DMA,
                   pltpu.VMEM_SHARED((N_PAD, HD), jnp.float32)],
)


def _split_stack_nodes(a):
    return a.reshape(N, 2, HD).transpose(1, 0, 2).reshape(2 * N, HD)


def _split_stack_edges(a):
    return a.reshape(E, 2, HD).transpose(1, 0, 2).reshape(2 * E, HD)


def _unsplit_nodes(a):
    return a.reshape(2, N_PAD, HD)[:, :N].transpose(1, 0, 2).reshape(N, D)


def kernel(x, edge_attr, Wn, bn, Wee, bee, Wae, bae, ln_g, ln_b,
           in_Wa, in_We1, in_We2, in_Wc, in_bc,
           out_Wa, out_We1, out_We2, out_Wc, out_bc,
           EWself, EWmsg, Eb, Wfin, bfin, G1, g1b, G2, g2b,
           edge_index, batch):
    u = edge_index[0]
    v = edge_index[1]
    u2 = jnp.concatenate([u, u + N])
    zeros_half = jnp.zeros((N_PAD, HD), jnp.float32)

    gkey = jax.random.key(12345)
    gcols = []
    for i in range(2 * NUM_LAYERS):
        uu = jax.random.uniform(jax.random.fold_in(gkey, i), (N, 2),
                                minval=1e-6, maxval=1.0 - 1e-6)
        g = -jnp.log(-jnp.log(uu))
        bc = in_bc if i % 2 == 0 else out_bc
        gcols.append(((bc[0] + g[:, 0]).reshape(N, 1),
                      (bc[1] + g[:, 1]).reshape(N, 1)))
    wi0 = in_Wc[:, 0].reshape(D, 1)
    wi1 = in_Wc[:, 1].reshape(D, 1)
    wo0 = out_Wc[:, 0].reshape(D, 1)
    wo1 = out_Wc[:, 1].reshape(D, 1)

    env, p_in, p_out = _edge_prologue(edge_attr, Wee, bee, Wae, bae,
                                      in_We1, in_We2, out_We1, out_We2)
    env2 = _split_stack_edges(env)
    p_in2 = _split_stack_edges(p_in)
    p_out2 = _split_stack_edges(p_out)

    hL = _node_prologue(x, Wn, bn, ln_g, ln_b)

    for i in range(NUM_LAYERS):
        hWa_in, hWa_out = _premul(hL, in_Wa, out_Wa)
        S_in = _unsplit_nodes(_sweep_gather_add()(
            u2, v, _split_stack_nodes(hWa_in), p_in2, zeros_half))
        S_out = _unsplit_nodes(_sweep_gather_add()(
            u2, v, _split_stack_nodes(hWa_out), p_out2, zeros_half))
        gi0, gi1 = gcols[2 * i]
        go0, go1 = gcols[2 * i + 1]
        b_in, b_out, hB = _gates(S_in, S_out, hL, wi0, wi1, wo0, wo1,
                                 gi0, gi1, go0, go1)
        b2 = jnp.concatenate([b_out.reshape(N), b_out.reshape(N)])
        T = _unsplit_nodes(_sweep_gather_env()(
            u2, v, _split_stack_nodes(hB), b2, env2, zeros_half))
        hL = _update(hL, T, b_in, EWself[i], EWmsg[i], Eb[i], ln_g, ln_b)

    hF, gate = _final(hL, Wfin, bfin, G1, g1b, G2, g2b)
    oh = (batch.reshape(N, 1) == jnp.arange(NG, dtype=batch.dtype).reshape(1, NG)
          ).astype(jnp.float32)
    return _pool(gate, hF, oh)


# speedup vs baseline: 1.1915x; 1.1915x over previous
import functools

import jax
import jax.numpy as jnp
from jax import lax
from jax.experimental import pallas as pl
from jax.experimental.pallas import tpu as pltpu
from jax.experimental.pallas import tpu_sc as plsc

N = 50000
E = 800000
D = 64
HD = 32
NG = 16
NUM_LAYERS = 3

NTILES = 16
N_PAD = 50176
NPT = N_PAD // NTILES
EPT = E // NTILES
CH = 80
NCHUNK = EPT // CH

BN = 2000
NBN = N // BN
BE = 8000
NBE = E // BE


def _ln_rows(t, g, b):
    mu = jnp.mean(t, axis=-1, keepdims=True)
    var = jnp.mean((t - mu) * (t - mu), axis=-1, keepdims=True)
    return (t - mu) * lax.rsqrt(var + 1e-5) * g + b


def _prologue_body(x_ref, wn_ref, bn_ref, lng_ref, lnb_ref, hl_ref):
    h = jnp.dot(x_ref[...], wn_ref[...], preferred_element_type=jnp.float32)
    h = jnp.maximum(h + bn_ref[...], 0.0)
    hl_ref[...] = _ln_rows(h, lng_ref[...], lnb_ref[...])


def _node_prologue(x, Wn, bn, ln_g, ln_b):
    return pl.pallas_call(
        _prologue_body,
        grid=(NBN,),
        in_specs=[
            pl.BlockSpec((BN, 153), lambda i: (i, 0)),
            pl.BlockSpec((153, D), lambda i: (0, 0)),
            pl.BlockSpec((1, D), lambda i: (0, 0)),
            pl.BlockSpec((1, D), lambda i: (0, 0)),
            pl.BlockSpec((1, D), lambda i: (0, 0)),
        ],
        out_specs=pl.BlockSpec((BN, D), lambda i: (i, 0)),
        out_shape=jax.ShapeDtypeStruct((N, D), jnp.float32),
    )(x, Wn, bn.reshape(1, D), ln_g.reshape(1, D), ln_b.reshape(1, D))


def _edge_prologue_body(ea_ref, wee_ref, bee_ref, wae_ref, bae_ref,
                        wi1_ref, wi2_ref, wo1_ref, wo2_ref,
                        env_ref, pin_ref, pout_ref):
    ea = ea_ref[...]
    env = jnp.maximum(jnp.dot(ea, wee_ref[...], preferred_element_type=jnp.float32) + bee_ref[...], 0.0)
    act = jnp.maximum(jnp.dot(ea, wae_ref[...], preferred_element_type=jnp.float32) + bae_ref[...], 0.0)
    env_ref[...] = env
    pin_ref[...] = (jnp.dot(env, wi1_ref[...], preferred_element_type=jnp.float32)
                    + jnp.dot(act, wi2_ref[...], preferred_element_type=jnp.float32))
    pout_ref[...] = (jnp.dot(env, wo1_ref[...], preferred_element_type=jnp.float32)
                     + jnp.dot(act, wo2_ref[...], preferred_element_type=jnp.float32))


def _edge_prologue(edge_attr, Wee, bee, Wae, bae, iW1, iW2, oW1, oW2):
    wspec = pl.BlockSpec((7, D), lambda i: (0, 0))
    dspec = pl.BlockSpec((D, D), lambda i: (0, 0))
    bspec = pl.BlockSpec((1, D), lambda i: (0, 0))
    espec = pl.BlockSpec((BE, D), lambda i: (i, 0))
    return pl.pallas_call(
        _edge_prologue_body,
        grid=(NBE,),
        in_specs=[pl.BlockSpec((BE, 7), lambda i: (i, 0)),
                  wspec, bspec, wspec, bspec, dspec, dspec, dspec, dspec],
        out_specs=[espec, espec, espec],
        out_shape=[jax.ShapeDtypeStruct((E, D), jnp.float32)] * 3,
    )(edge_attr, Wee, bee.reshape(1, D), Wae, bae.reshape(1, D), iW1, iW2, oW1, oW2)


def _premul_body(hl_ref, wia_ref, woa_ref, hin_ref, hout_ref):
    hl = hl_ref[...]
    hin_ref[...] = jnp.dot(hl, wia_ref[...], preferred_element_type=jnp.float32)
    hout_ref[...] = jnp.dot(hl, woa_ref[...], preferred_element_type=jnp.float32)


def _premul(hL, in_Wa, out_Wa):
    nspec = pl.BlockSpec((BN, D), lambda i: (i, 0))
    dspec = pl.BlockSpec((D, D), lambda i: (0, 0))
    return pl.pallas_call(
        _premul_body,
        grid=(NBN,),
        in_specs=[nspec, dspec, dspec],
        out_specs=[nspec, nspec],
        out_shape=[jax.ShapeDtypeStruct((N, D), jnp.float32)] * 2,
    )(hL, in_Wa, out_Wa)


def _gates_body(sin_ref, sout_ref, cin_ref, cout_ref, hl_ref,
                wi0_ref, wi1_ref, wo0_ref, wo1_ref,
                gi0_ref, gi1_ref, go0_ref, go1_ref,
                bin_ref, bout_ref, hb_ref):
    a_in = jnp.maximum(sin_ref[...] + cin_ref[...], 0.0)
    a_out = jnp.maximum(sout_ref[...] + cout_ref[...], 0.0)
    li0 = jnp.dot(a_in, wi0_ref[...], preferred_element_type=jnp.float32) + gi0_ref[...]
    li1 = jnp.dot(a_in, wi1_ref[...], preferred_element_type=jnp.float32) + gi1_ref[...]
    lo0 = jnp.dot(a_out, wo0_ref[...], preferred_element_type=jnp.float32) + go0_ref[...]
    lo1 = jnp.dot(a_out, wo1_ref[...], preferred_element_type=jnp.float32) + go1_ref[...]
    b_in = (li0 >= li1).astype(jnp.float32)
    b_out = (lo0 >= lo1).astype(jnp.float32)
    bin_ref[...] = b_in
    bout_ref[...] = b_out
    hb_ref[...] = b_out * hl_ref[...]


def _gates(S_in, S_out, C_in, C_out, hL, wi0, wi1, wo0, wo1,
           gi0, gi1, go0, go1):
    nspec = pl.BlockSpec((BN, D), lambda i: (i, 0))
    cspec = pl.BlockSpec((D, 1), lambda i: (0, 0))
    vspec = pl.BlockSpec((BN, 1), lambda i: (i, 0))
    return pl.pallas_call(
        _gates_body,
        grid=(NBN,),
        in_specs=[nspec, nspec, nspec, nspec, nspec,
                  cspec, cspec, cspec, cspec,
                  vspec, vspec, vspec, vspec],
        out_specs=[vspec, vspec, nspec],
        out_shape=[jax.ShapeDtypeStruct((N, 1), jnp.float32),
                   jax.ShapeDtypeStruct((N, 1), jnp.float32),
                   jax.ShapeDtypeStruct((N, D), jnp.float32)],
    )(S_in, S_out, C_in, C_out, hL, wi0, wi1, wo0, wo1, gi0, gi1, go0, go1)


def _update_body(hl_ref, t_ref, bin_ref, ws_ref, wm_ref, eb_ref,
                 lng_ref, lnb_ref, out_ref):
    hl = hl_ref[...]
    agg = bin_ref[...] * t_ref[...]
    o = (jnp.dot(hl, ws_ref[...], preferred_element_type=jnp.float32)
         + jnp.dot(agg, wm_ref[...], preferred_element_type=jnp.float32)
         + eb_ref[...])
    r = hl + jnp.maximum(o, 0.0)
    out_ref[...] = _ln_rows(r, lng_ref[...], lnb_ref[...])


def _update(hL, T, b_in, Ws, Wm, Ebi, ln_g, ln_b):
    nspec = pl.BlockSpec((BN, D), lambda i: (i, 0))
    dspec = pl.BlockSpec((D, D), lambda i: (0, 0))
    bspec = pl.BlockSpec((1, D), lambda i: (0, 0))
    return pl.pallas_call(
        _update_body,
        grid=(NBN,),
        in_specs=[nspec, nspec, pl.BlockSpec((BN, 1), lambda i: (i, 0)),
                  dspec, dspec, bspec, bspec, bspec],
        out_specs=nspec,
        out_shape=jax.ShapeDtypeStruct((N, D), jnp.float32),
    )(hL, T, b_in, Ws, Wm, Ebi.reshape(1, D), ln_g.reshape(1, D), ln_b.reshape(1, D))


def _final_body(hl_ref, wf_ref, bf_ref, g1_ref, g1b_ref, g2_ref, g2b_ref,
                hf_ref, gate_ref):
    hf = jnp.dot(hl_ref[...], wf_ref[...], preferred_element_type=jnp.float32) + bf_ref[...]
    hf_ref[...] = hf
    t = jnp.maximum(jnp.dot(hf, g1_ref[...], preferred_element_type=jnp.float32) + g1b_ref[...], 0.0)
    gate_ref[...] = jnp.dot(t, g2_ref[...], preferred_element_type=jnp.float32) + g2b_ref[...]


def _final(hL, Wfin, bfin, G1, g1b, G2, g2b):
    nspec = pl.BlockSpec((BN, D), lambda i: (i, 0))
    dspec = pl.BlockSpec((D, D), lambda i: (0, 0))
    bspec = pl.BlockSpec((1, D), lambda i: (0, 0))
    return pl.pallas_call(
        _final_body,
        grid=(NBN,),
        in_specs=[nspec, dspec, bspec, dspec, bspec,
                  pl.BlockSpec((D, 1), lambda i: (0, 0)),
                  pl.BlockSpec((1, 1), lambda i: (0, 0))],
        out_specs=[nspec, pl.BlockSpec((BN, 1), lambda i: (i, 0))],
        out_shape=[jax.ShapeDtypeStruct((N, D), jnp.float32),
                   jax.ShapeDtypeStruct((N, 1), jnp.float32)],
    )(hL, Wfin, bfin.reshape(1, D), G1, g1b.reshape(1, D), G2,
      g2b.reshape(1, 1))


def _pool_body(gate_ref, hf_ref, oh_ref, out_ref, smax, sden, snum):
    p = pl.program_id(0)
    j = pl.program_id(1)
    oh = oh_ref[...]
    gate = gate_ref[...]

    @pl.when(jnp.logical_and(p == 0, j == 0))
    def _():
        smax[...] = jnp.full((8, NG), -jnp.inf, jnp.float32)

    @pl.when(p == 0)
    def _():
        m = jnp.max(jnp.where(oh > 0.0, gate, -jnp.inf), axis=0, keepdims=True)
        smax[0:1, :] = jnp.maximum(smax[0:1, :], m)

    @pl.when(jnp.logical_and(p == 1, j == 0))
    def _():
        sden[...] = jnp.zeros((8, NG), jnp.float32)
        snum[...] = jnp.zeros((NG, D), jnp.float32)

    @pl.when(p == 1)
    def _():
        gmax_node = jnp.sum(oh * smax[0:1, :], axis=1, keepdims=True)
        ex = jnp.exp(gate - gmax_node)
        sden[0:1, :] = sden[0:1, :] + jnp.sum(oh * ex, axis=0, keepdims=True)
        snum[...] = snum[...] + lax.dot_general(
            oh, ex * hf_ref[...], (((0,), (0,)), ((), ())),
            preferred_element_type=jnp.float32)

    @pl.when(jnp.logical_and(p == 1, j == NBN - 1))
    def _():
        den_col = jnp.reshape(sden[0:1, :], (NG, 1))
        out_ref[...] = snum[...] / (den_col + 1e-16)


def _pool(gate, hF, oh):
    return pl.pallas_call(
        _pool_body,
        grid=(2, NBN),
        in_specs=[pl.BlockSpec((BN, 1), lambda p, j: (j, 0)),
                  pl.BlockSpec((BN, D), lambda p, j: (j, 0)),
                  pl.BlockSpec((BN, NG), lambda p, j: (j, 0))],
        out_specs=pl.BlockSpec((NG, D), lambda p, j: (0, 0)),
        out_shape=jax.ShapeDtypeStruct((NG, D), jnp.float32),
        scratch_shapes=[pltpu.VMEM((8, NG), jnp.float32),
                        pltpu.VMEM((8, NG), jnp.float32),
                        pltpu.VMEM((NG, D), jnp.float32)],
    )(gate, hF, oh)


_MESH = plsc.VectorSubcoreMesh(core_axis_name="c", subcore_axis_name="s")
_SC_PARAMS = pltpu.CompilerParams(use_tc_tiling_on_sc=False,
                                  needs_layout_passes=False)


def _sweep_linear_kernel(v_hbm, rows2_hbm, zeros_hbm, out_hbm,
                         vidx_v, rows_v, shared):
    c = lax.axis_index("c")
    s = lax.axis_index("s")
    pltpu.sync_copy(zeros_hbm.at[pl.ds(s * NPT, NPT)],
                    shared.at[pl.ds(s * NPT, NPT)])
    plsc.subcore_barrier()

    def chunk(i, carry):
        base = s * EPT + i * CH
        pltpu.sync_copy(v_hbm.at[pl.ds(base, CH)], vidx_v)
        pltpu.sync_copy(rows2_hbm.at[pl.ds(c * E + base, CH)], rows_v)
        pltpu.sync_copy(rows_v, shared.at[vidx_v], add=True)
        return carry

    lax.fori_loop(0, NCHUNK, chunk, 0)
    plsc.subcore_barrier()
    pltpu.sync_copy(shared.at[pl.ds(s * NPT, NPT)],
                    out_hbm.at[pl.ds(c * N_PAD + s * NPT, NPT)])


def _sweep_gather_kernel(u2_hbm, v_hbm, table_hbm, zeros_hbm, out_hbm,
                         uidx_v, vidx_v, rows_v, sem, shared):
    c = lax.axis_index("c")
    s = lax.axis_index("s")
    pltpu.sync_copy(zeros_hbm.at[pl.ds(s * NPT, NPT)],
                    shared.at[pl.ds(s * NPT, NPT)])
    plsc.subcore_barrier()

    def chunk(i, carry):
        base = s * EPT + i * CH
        pltpu.sync_copy(u2_hbm.at[pl.ds(c * E + base, CH)], uidx_v)
        pltpu.sync_copy(v_hbm.at[pl.ds(base, CH)], vidx_v)
        pltpu.async_copy(table_hbm.at[uidx_v], rows_v, sem).wait()
        pltpu.sync_copy(rows_v, shared.at[vidx_v], add=True)
        return carry

    lax.fori_loop(0, NCHUNK, chunk, 0)
    plsc.subcore_barrier()
    pltpu.sync_copy(shared.at[pl.ds(s * NPT, NPT)],
                    out_hbm.at[pl.ds(c * N_PAD + s * NPT, NPT)])


def _sweep_msg_kernel(u2_hbm, v_hbm, table_hbm, b2_hbm, env2_hbm,
                      zeros_hbm, out_hbm,
                      uidx_v, vidx_v, vidx2_v, rows_v, env_v, bval_v,
                      sem, semb, shared):
    c = lax.axis_index("c")
    s = lax.axis_index("s")
    pltpu.sync_copy(zeros_hbm.at[pl.ds(s * NPT, NPT)],
                    shared.at[pl.ds(s * NPT, NPT)])
    plsc.subcore_barrier()
    trash = jnp.full((16,), N, jnp.int32)

    def chunk(i, carry):
        base = s * EPT + i * CH
        pltpu.sync_copy(u2_hbm.at[pl.ds(c * E + base, CH)], uidx_v)
        pltpu.sync_copy(v_hbm.at[pl.ds(base, CH)], vidx_v)
        pltpu.sync_copy(env2_hbm.at[pl.ds(c * E + base, CH)], env_v)
        cp1 = pltpu.async_copy(table_hbm.at[uidx_v], rows_v, sem)
        cp2 = pltpu.async_copy(b2_hbm.at[uidx_v], bval_v, semb)
        cp1.wait()
        cp2.wait()
        for k in range(CH // 16):
            b = bval_v[pl.ds(16 * k, 16)]
            vi = vidx_v[pl.ds(16 * k, 16)]
            vidx2_v[pl.ds(16 * k, 16)] = jnp.where(b >= 0.5, vi, trash)
        pltpu.sync_copy(rows_v, shared.at[vidx_v], add=True)
        pltpu.sync_copy(env_v, shared.at[vidx2_v], add=True)
        return carry

    lax.fori_loop(0, NCHUNK, chunk, 0)
    plsc.subcore_barrier()
    pltpu.sync_copy(shared.at[pl.ds(s * NPT, NPT)],
                    out_hbm.at[pl.ds(c * N_PAD + s * NPT, NPT)])


_sweep_linear = functools.partial(
    pl.kernel, _sweep_linear_kernel, mesh=_MESH, compiler_params=_SC_PARAMS,
    out_type=jax.ShapeDtypeStruct((2 * N_PAD, HD), jnp.float32),
    scratch_types=[pltpu.VMEM((CH,), jnp.int32),
                   pltpu.VMEM((CH, HD), jnp.float32),
                   pltpu.VMEM_SHARED((N_PAD, HD), jnp.float32)],
)

_sweep_gather = functools.partial(
    pl.kernel, _sweep_gather_kernel, mesh=_MESH, compiler_params=_SC_PARAMS,
    out_type=jax.ShapeDtypeStruct((2 * N_PAD, HD), jnp.float32),
    scratch_types=[pltpu.VMEM((CH,), jnp.int32),
                   pltpu.VMEM((CH,), jnp.int32),
                   pltpu.VMEM((CH, HD), jnp.float32),
                   pltpu.SemaphoreType.DMA,
                   pltpu.VMEM_SHARED((N_PAD, HD), jnp.float32)],
)

_sweep_msg = functools.partial(
    pl.kernel, _sweep_msg_kernel, mesh=_MESH, compiler_params=_SC_PARAMS,
    out_type=jax.ShapeDtypeStruct((2 * N_PAD, HD), jnp.float32),
    scratch_types=[pltpu.VMEM((CH,), jnp.int32),
                   pltpu.VMEM((CH,), jnp.int32),
                   pltpu.VMEM((CH,), jnp.int32),
                   pltpu.VMEM((CH, HD), jnp.float32),
                   pltpu.VMEM((CH, HD), jnp.float32),
                   pltpu.VMEM((CH,), jnp.float32),
                   pltpu.SemaphoreType.DMA,
                   pltpu.SemaphoreType.DMA,
                   pltpu.VMEM_SHARED((N_PAD, HD), jnp.float32)],
)


def _split_stack_nodes(a):
    return a.reshape(N, 2, HD).transpose(1, 0, 2).reshape(2 * N, HD)


def _split_stack_edges(a):
    return a.reshape(E, 2, HD).transpose(1, 0, 2).reshape(2 * E, HD)


def _unsplit_nodes(a):
    return a.reshape(2, N_PAD, HD)[:, :N].transpose(1, 0, 2).reshape(N, D)


def kernel(x, edge_attr, Wn, bn, Wee, bee, Wae, bae, ln_g, ln_b,
           in_Wa, in_We1, in_We2, in_Wc, in_bc,
           out_Wa, out_We1, out_We2, out_Wc, out_bc,
           EWself, EWmsg, Eb, Wfin, bfin, G1, g1b, G2, g2b,
           edge_index, batch):
    u = edge_index[0]
    v = edge_index[1]
    u2 = jnp.concatenate([u, u + N])
    zeros_half = jnp.zeros((N_PAD, HD), jnp.float32)

    gkey = jax.random.key(12345)
    gcols = []
    for i in range(2 * NUM_LAYERS):
        uu = jax.random.uniform(jax.random.fold_in(gkey, i), (N, 2),
                                minval=1e-6, maxval=1.0 - 1e-6)
        g = -jnp.log(-jnp.log(uu))
        bc = in_bc if i % 2 == 0 else out_bc
        gcols.append(((bc[0] + g[:, 0]).reshape(N, 1),
                      (bc[1] + g[:, 1]).reshape(N, 1)))
    wi0 = in_Wc[:, 0].reshape(D, 1)
    wi1 = in_Wc[:, 1].reshape(D, 1)
    wo0 = out_Wc[:, 0].reshape(D, 1)
    wo1 = out_Wc[:, 1].reshape(D, 1)

    env, p_in, p_out = _edge_prologue(edge_attr, Wee, bee, Wae, bae,
                                      in_We1, in_We2, out_We1, out_We2)
    env2 = _split_stack_edges(env)
    C_in = _unsplit_nodes(_sweep_linear()(v, _split_stack_edges(p_in), zeros_half))
    C_out = _unsplit_nodes(_sweep_linear()(v, _split_stack_edges(p_out), zeros_half))

    hL = _node_prologue(x, Wn, bn, ln_g, ln_b)

    for i in range(NUM_LAYERS):
        hWa_in, hWa_out = _premul(hL, in_Wa, out_Wa)
        S_in = _unsplit_nodes(_sweep_gather()(
            u2, v, _split_stack_nodes(hWa_in), zeros_half))
        S_out = _unsplit_nodes(_sweep_gather()(
            u2, v, _split_stack_nodes(hWa_out), zeros_half))
        gi0, gi1 = gcols[2 * i]
        go0, go1 = gcols[2 * i + 1]
        b_in, b_out, hB = _gates(S_in, S_out, C_in, C_out, hL,
                                 wi0, wi1, wo0, wo1, gi0, gi1, go0, go1)
        b2 = jnp.concatenate([b_out.reshape(N), b_out.reshape(N)])
        T = _unsplit_nodes(_sweep_msg()(
            u2, v, _split_stack_nodes(hB), b2, env2, zeros_half))
        hL = _update(hL, T, b_in, EWself[i], EWmsg[i], Eb[i], ln_g, ln_b)

    hF, gate = _final(hL, Wfin, bfin, G1, g1b, G2, g2b)
    oh = (batch.reshape(N, 1) == jnp.arange(NG, dtype=batch.dtype).reshape(1, NG)
          ).astype(jnp.float32)
    return _pool(gate, hF, oh)


# speedup vs baseline: 1.3923x; 1.1685x over previous
import functools

import jax
import jax.numpy as jnp
from jax import lax
from jax.experimental import pallas as pl
from jax.experimental.pallas import tpu as pltpu
from jax.experimental.pallas import tpu_sc as plsc

N = 50000
E = 800000
D = 64
HD = 32
NG = 16
NUM_LAYERS = 3

NTILES = 16
N_PAD = 50176
NPT = N_PAD // NTILES
EPT = E // NTILES
CH = 80
NCHUNK = EPT // CH

BN = 2000
NBN = N // BN
BE = 8000
NBE = E // BE


def _ln_rows(t, g, b):
    mu = jnp.mean(t, axis=-1, keepdims=True)
    var = jnp.mean((t - mu) * (t - mu), axis=-1, keepdims=True)
    return (t - mu) * lax.rsqrt(var + 1e-5) * g + b


def _prologue_body(x_ref, wn_ref, bn_ref, lng_ref, lnb_ref, hl_ref):
    h = jnp.dot(x_ref[...], wn_ref[...], preferred_element_type=jnp.float32)
    h = jnp.maximum(h + bn_ref[...], 0.0)
    hl_ref[...] = _ln_rows(h, lng_ref[...], lnb_ref[...])


def _node_prologue(x, Wn, bn, ln_g, ln_b):
    return pl.pallas_call(
        _prologue_body,
        grid=(NBN,),
        in_specs=[
            pl.BlockSpec((BN, 153), lambda i: (i, 0)),
            pl.BlockSpec((153, D), lambda i: (0, 0)),
            pl.BlockSpec((1, D), lambda i: (0, 0)),
            pl.BlockSpec((1, D), lambda i: (0, 0)),
            pl.BlockSpec((1, D), lambda i: (0, 0)),
        ],
        out_specs=pl.BlockSpec((BN, D), lambda i: (i, 0)),
        out_shape=jax.ShapeDtypeStruct((N, D), jnp.float32),
    )(x, Wn, bn.reshape(1, D), ln_g.reshape(1, D), ln_b.reshape(1, D))


def _edge_prologue_body(ea_ref, wee_ref, bee_ref, wae_ref, bae_ref,
                        wi1_ref, wi2_ref, wo1_ref, wo2_ref,
                        env_ref, pin_ref, pout_ref):
    ea = ea_ref[...]
    env = jnp.maximum(jnp.dot(ea, wee_ref[...], preferred_element_type=jnp.float32) + bee_ref[...], 0.0)
    act = jnp.maximum(jnp.dot(ea, wae_ref[...], preferred_element_type=jnp.float32) + bae_ref[...], 0.0)
    env_ref[...] = env
    pin_ref[...] = (jnp.dot(env, wi1_ref[...], preferred_element_type=jnp.float32)
                    + jnp.dot(act, wi2_ref[...], preferred_element_type=jnp.float32))
    pout_ref[...] = (jnp.dot(env, wo1_ref[...], preferred_element_type=jnp.float32)
                     + jnp.dot(act, wo2_ref[...], preferred_element_type=jnp.float32))


def _edge_prologue(edge_attr, Wee, bee, Wae, bae, iW1, iW2, oW1, oW2):
    wspec = pl.BlockSpec((7, D), lambda i: (0, 0))
    dspec = pl.BlockSpec((D, D), lambda i: (0, 0))
    bspec = pl.BlockSpec((1, D), lambda i: (0, 0))
    espec = pl.BlockSpec((BE, D), lambda i: (i, 0))
    return pl.pallas_call(
        _edge_prologue_body,
        grid=(NBE,),
        in_specs=[pl.BlockSpec((BE, 7), lambda i: (i, 0)),
                  wspec, bspec, wspec, bspec, dspec, dspec, dspec, dspec],
        out_specs=[espec, espec, espec],
        out_shape=[jax.ShapeDtypeStruct((E, D), jnp.float32)] * 3,
    )(edge_attr, Wee, bee.reshape(1, D), Wae, bae.reshape(1, D), iW1, iW2, oW1, oW2)


def _premul_body(hl_ref, wia_ref, woa_ref, hin_ref, hout_ref):
    hl = hl_ref[...]
    hin_ref[...] = jnp.dot(hl, wia_ref[...], preferred_element_type=jnp.float32)
    hout_ref[...] = jnp.dot(hl, woa_ref[...], preferred_element_type=jnp.float32)


def _premul(hL, in_Wa, out_Wa):
    nspec = pl.BlockSpec((BN, D), lambda i: (i, 0))
    dspec = pl.BlockSpec((D, D), lambda i: (0, 0))
    return pl.pallas_call(
        _premul_body,
        grid=(NBN,),
        in_specs=[nspec, dspec, dspec],
        out_specs=[nspec, nspec],
        out_shape=[jax.ShapeDtypeStruct((N, D), jnp.float32)] * 2,
    )(hL, in_Wa, out_Wa)


def _gates_body(sin_ref, sout_ref, cin_ref, cout_ref, hl_ref,
                wi0_ref, wi1_ref, wo0_ref, wo1_ref,
                gi0_ref, gi1_ref, go0_ref, go1_ref,
                bin_ref, bout_ref, hb_ref):
    a_in = jnp.maximum(sin_ref[...] + cin_ref[...], 0.0)
    a_out = jnp.maximum(sout_ref[...] + cout_ref[...], 0.0)
    li0 = jnp.dot(a_in, wi0_ref[...], preferred_element_type=jnp.float32) + gi0_ref[...]
    li1 = jnp.dot(a_in, wi1_ref[...], preferred_element_type=jnp.float32) + gi1_ref[...]
    lo0 = jnp.dot(a_out, wo0_ref[...], preferred_element_type=jnp.float32) + go0_ref[...]
    lo1 = jnp.dot(a_out, wo1_ref[...], preferred_element_type=jnp.float32) + go1_ref[...]
    b_in = (li0 >= li1).astype(jnp.float32)
    b_out = (lo0 >= lo1).astype(jnp.float32)
    bin_ref[...] = b_in
    bout_ref[...] = b_out
    hb_ref[...] = b_out * hl_ref[...]


def _gates(S_in, S_out, C_in, C_out, hL, wi0, wi1, wo0, wo1,
           gi0, gi1, go0, go1):
    nspec = pl.BlockSpec((BN, D), lambda i: (i, 0))
    cspec = pl.BlockSpec((D, 1), lambda i: (0, 0))
    vspec = pl.BlockSpec((BN, 1), lambda i: (i, 0))
    return pl.pallas_call(
        _gates_body,
        grid=(NBN,),
        in_specs=[nspec, nspec, nspec, nspec, nspec,
                  cspec, cspec, cspec, cspec,
                  vspec, vspec, vspec, vspec],
        out_specs=[vspec, vspec, nspec],
        out_shape=[jax.ShapeDtypeStruct((N, 1), jnp.float32),
                   jax.ShapeDtypeStruct((N, 1), jnp.float32),
                   jax.ShapeDtypeStruct((N, D), jnp.float32)],
    )(S_in, S_out, C_in, C_out, hL, wi0, wi1, wo0, wo1, gi0, gi1, go0, go1)


def _update_body(hl_ref, t_ref, bin_ref, ws_ref, wm_ref, eb_ref,
                 lng_ref, lnb_ref, out_ref):
    hl = hl_ref[...]
    agg = bin_ref[...] * t_ref[...]
    o = (jnp.dot(hl, ws_ref[...], preferred_element_type=jnp.float32)
         + jnp.dot(agg, wm_ref[...], preferred_element_type=jnp.float32)
         + eb_ref[...])
    r = hl + jnp.maximum(o, 0.0)
    out_ref[...] = _ln_rows(r, lng_ref[...], lnb_ref[...])


def _update(hL, T, b_in, Ws, Wm, Ebi, ln_g, ln_b):
    nspec = pl.BlockSpec((BN, D), lambda i: (i, 0))
    dspec = pl.BlockSpec((D, D), lambda i: (0, 0))
    bspec = pl.BlockSpec((1, D), lambda i: (0, 0))
    return pl.pallas_call(
        _update_body,
        grid=(NBN,),
        in_specs=[nspec, nspec, pl.BlockSpec((BN, 1), lambda i: (i, 0)),
                  dspec, dspec, bspec, bspec, bspec],
        out_specs=nspec,
        out_shape=jax.ShapeDtypeStruct((N, D), jnp.float32),
    )(hL, T, b_in, Ws, Wm, Ebi.reshape(1, D), ln_g.reshape(1, D), ln_b.reshape(1, D))


def _final_body(hl_ref, wf_ref, bf_ref, g1_ref, g1b_ref, g2_ref, g2b_ref,
                hf_ref, gate_ref):
    hf = jnp.dot(hl_ref[...], wf_ref[...], preferred_element_type=jnp.float32) + bf_ref[...]
    hf_ref[...] = hf
    t = jnp.maximum(jnp.dot(hf, g1_ref[...], preferred_element_type=jnp.float32) + g1b_ref[...], 0.0)
    gate_ref[...] = jnp.dot(t, g2_ref[...], preferred_element_type=jnp.float32) + g2b_ref[...]


def _final(hL, Wfin, bfin, G1, g1b, G2, g2b):
    nspec = pl.BlockSpec((BN, D), lambda i: (i, 0))
    dspec = pl.BlockSpec((D, D), lambda i: (0, 0))
    bspec = pl.BlockSpec((1, D), lambda i: (0, 0))
    return pl.pallas_call(
        _final_body,
        grid=(NBN,),
        in_specs=[nspec, dspec, bspec, dspec, bspec,
                  pl.BlockSpec((D, 1), lambda i: (0, 0)),
                  pl.BlockSpec((1, 1), lambda i: (0, 0))],
        out_specs=[nspec, pl.BlockSpec((BN, 1), lambda i: (i, 0))],
        out_shape=[jax.ShapeDtypeStruct((N, D), jnp.float32),
                   jax.ShapeDtypeStruct((N, 1), jnp.float32)],
    )(hL, Wfin, bfin.reshape(1, D), G1, g1b.reshape(1, D), G2,
      g2b.reshape(1, 1))


def _pool_body(gate_ref, hf_ref, oh_ref, out_ref, smax, sden, snum):
    p = pl.program_id(0)
    j = pl.program_id(1)
    oh = oh_ref[...]
    gate = gate_ref[...]

    @pl.when(jnp.logical_and(p == 0, j == 0))
    def _():
        smax[...] = jnp.full((8, NG), -jnp.inf, jnp.float32)

    @pl.when(p == 0)
    def _():
        m = jnp.max(jnp.where(oh > 0.0, gate, -jnp.inf), axis=0, keepdims=True)
        smax[0:1, :] = jnp.maximum(smax[0:1, :], m)

    @pl.when(jnp.logical_and(p == 1, j == 0))
    def _():
        sden[...] = jnp.zeros((8, NG), jnp.float32)
        snum[...] = jnp.zeros((NG, D), jnp.float32)

    @pl.when(p == 1)
    def _():
        gmax_node = jnp.sum(oh * smax[0:1, :], axis=1, keepdims=True)
        ex = jnp.exp(gate - gmax_node)
        sden[0:1, :] = sden[0:1, :] + jnp.sum(oh * ex, axis=0, keepdims=True)
        snum[...] = snum[...] + lax.dot_general(
            oh, ex * hf_ref[...], (((0,), (0,)), ((), ())),
            preferred_element_type=jnp.float32)

    @pl.when(jnp.logical_and(p == 1, j == NBN - 1))
    def _():
        den_col = jnp.reshape(sden[0:1, :], (NG, 1))
        out_ref[...] = snum[...] / (den_col + 1e-16)


def _pool(gate, hF, oh):
    return pl.pallas_call(
        _pool_body,
        grid=(2, NBN),
        in_specs=[pl.BlockSpec((BN, 1), lambda p, j: (j, 0)),
                  pl.BlockSpec((BN, D), lambda p, j: (j, 0)),
                  pl.BlockSpec((BN, NG), lambda p, j: (j, 0))],
        out_specs=pl.BlockSpec((NG, D), lambda p, j: (0, 0)),
        out_shape=jax.ShapeDtypeStruct((NG, D), jnp.float32),
        scratch_shapes=[pltpu.VMEM((8, NG), jnp.float32),
                        pltpu.VMEM((8, NG), jnp.float32),
                        pltpu.VMEM((NG, D), jnp.float32)],
    )(gate, hF, oh)


_MESH = plsc.VectorSubcoreMesh(core_axis_name="c", subcore_axis_name="s")
_SC_PARAMS = pltpu.CompilerParams(use_tc_tiling_on_sc=False,
                                  needs_layout_passes=False)


def _sweep_linear_kernel(v_hbm, rows2_hbm, zeros_hbm, out_hbm,
                         vidx_v, rows_v, shared):
    c = lax.axis_index("c")
    s = lax.axis_index("s")
    pltpu.sync_copy(zeros_hbm.at[pl.ds(s * NPT, NPT)],
                    shared.at[pl.ds(s * NPT, NPT)])
    plsc.subcore_barrier()

    def chunk(i, carry):
        base = s * EPT + i * CH
        pltpu.sync_copy(v_hbm.at[pl.ds(base, CH)], vidx_v)
        pltpu.sync_copy(rows2_hbm.at[pl.ds(c * E + base, CH)], rows_v)
        pltpu.sync_copy(rows_v, shared.at[vidx_v], add=True)
        return carry

    lax.fori_loop(0, NCHUNK, chunk, 0)
    plsc.subcore_barrier()
    pltpu.sync_copy(shared.at[pl.ds(s * NPT, NPT)],
                    out_hbm.at[pl.ds(c * N_PAD + s * NPT, NPT)])


def _sweep_gather_kernel(u2_hbm, v_hbm, table_hbm, zeros_hbm, out_hbm,
                         uidx0_v, uidx1_v, vidx0_v, vidx1_v,
                         rows0_v, rows1_v, sem0, sem1, shared):
    c = lax.axis_index("c")
    s = lax.axis_index("s")
    pltpu.sync_copy(zeros_hbm.at[pl.ds(s * NPT, NPT)],
                    shared.at[pl.ds(s * NPT, NPT)])
    plsc.subcore_barrier()
    ebase = s * EPT

    def fire(i, uidx_v, vidx_v, rows_v, sem):
        pltpu.sync_copy(u2_hbm.at[pl.ds(c * E + ebase + i * CH, CH)], uidx_v)
        pltpu.sync_copy(v_hbm.at[pl.ds(ebase + i * CH, CH)], vidx_v)
        pltpu.async_copy(table_hbm.at[uidx_v], rows_v, sem)

    def drain(uidx_v, vidx_v, rows_v, sem):
        pltpu.make_async_copy(table_hbm.at[uidx_v], rows_v, sem).wait()
        pltpu.sync_copy(rows_v, shared.at[vidx_v], add=True)

    fire(0, uidx0_v, vidx0_v, rows0_v, sem0)

    def pair(i2, carry):
        a = 2 * i2
        fire(a + 1, uidx1_v, vidx1_v, rows1_v, sem1)
        drain(uidx0_v, vidx0_v, rows0_v, sem0)
        fire(a + 2, uidx0_v, vidx0_v, rows0_v, sem0)
        drain(uidx1_v, vidx1_v, rows1_v, sem1)
        return carry

    lax.fori_loop(0, (NCHUNK - 1) // 2, pair, 0)
    drain(uidx0_v, vidx0_v, rows0_v, sem0)
    plsc.subcore_barrier()
    pltpu.sync_copy(shared.at[pl.ds(s * NPT, NPT)],
                    out_hbm.at[pl.ds(c * N_PAD + s * NPT, NPT)])


def _sweep_msg_kernel(u2_hbm, v_hbm, table_hbm, b2_hbm, env2_hbm,
                      zeros_hbm, out_hbm,
                      uidx_v, vidx_v, vidx2_v, rows_v, env_v, bval_v,
                      sem, semb, shared):
    c = lax.axis_index("c")
    s = lax.axis_index("s")
    pltpu.sync_copy(zeros_hbm.at[pl.ds(s * NPT, NPT)],
                    shared.at[pl.ds(s * NPT, NPT)])
    plsc.subcore_barrier()
    trash = jnp.full((16,), N, jnp.int32)

    def chunk(i, carry):
        base = s * EPT + i * CH
        pltpu.sync_copy(u2_hbm.at[pl.ds(c * E + base, CH)], uidx_v)
        pltpu.sync_copy(v_hbm.at[pl.ds(base, CH)], vidx_v)
        pltpu.sync_copy(env2_hbm.at[pl.ds(c * E + base, CH)], env_v)
        cp1 = pltpu.async_copy(table_hbm.at[uidx_v], rows_v, sem)
        cp2 = pltpu.async_copy(b2_hbm.at[uidx_v], bval_v, semb)
        cp1.wait()
        cp2.wait()
        for k in range(CH // 16):
            b = bval_v[pl.ds(16 * k, 16)]
            vi = vidx_v[pl.ds(16 * k, 16)]
            vidx2_v[pl.ds(16 * k, 16)] = jnp.where(b >= 0.5, vi, trash)
        pltpu.sync_copy(rows_v, shared.at[vidx_v], add=True)
        pltpu.sync_copy(env_v, shared.at[vidx2_v], add=True)
        return carry

    lax.fori_loop(0, NCHUNK, chunk, 0)
    plsc.subcore_barrier()
    pltpu.sync_copy(shared.at[pl.ds(s * NPT, NPT)],
                    out_hbm.at[pl.ds(c * N_PAD + s * NPT, NPT)])


_sweep_linear = functools.partial(
    pl.kernel, _sweep_linear_kernel, mesh=_MESH, compiler_params=_SC_PARAMS,
    out_type=jax.ShapeDtypeStruct((2 * N_PAD, HD), jnp.float32),
    scratch_types=[pltpu.VMEM((CH,), jnp.int32),
                   pltpu.VMEM((CH, HD), jnp.float32),
                   pltpu.VMEM_SHARED((N_PAD, HD), jnp.float32)],
)

_sweep_gather = functools.partial(
    pl.kernel, _sweep_gather_kernel, mesh=_MESH, compiler_params=_SC_PARAMS,
    out_type=jax.ShapeDtypeStruct((2 * N_PAD, HD), jnp.float32),
    scratch_types=[pltpu.VMEM((CH,), jnp.int32),
                   pltpu.VMEM((CH,), jnp.int32),
                   pltpu.VMEM((CH,), jnp.int32),
                   pltpu.VMEM((CH,), jnp.int32),
                   pltpu.VMEM((CH, HD), jnp.float32),
                   pltpu.VMEM((CH, HD), jnp.float32),
                   pltpu.SemaphoreType.DMA,
                   pltpu.SemaphoreType.DMA,
                   pltpu.VMEM_SHARED((N_PAD, HD), jnp.float32)],
)

_sweep_msg = functools.partial(
    pl.kernel, _sweep_msg_kernel, mesh=_MESH, compiler_params=_SC_PARAMS,
    out_type=jax.ShapeDtypeStruct((2 * N_PAD, HD), jnp.float32),
    scratch_types=[pltpu.VMEM((CH,), jnp.int32),
                   pltpu.VMEM((CH,), jnp.int32),
                   pltpu.VMEM((CH,), jnp.int32),
                   pltpu.VMEM((CH, HD), jnp.float32),
                   pltpu.VMEM((CH, HD), jnp.float32),
                   pltpu.VMEM((CH,), jnp.float32),
                   pltpu.SemaphoreType.DMA,
                   pltpu.SemaphoreType.DMA,
                   pltpu.VMEM_SHARED((N_PAD, HD), jnp.float32)],
)


def _split_stack_nodes(a):
    return a.reshape(N, 2, HD).transpose(1, 0, 2).reshape(2 * N, HD)


def _split_stack_edges(a):
    return a.reshape(E, 2, HD).transpose(1, 0, 2).reshape(2 * E, HD)


def _unsplit_nodes(a):
    return a.reshape(2, N_PAD, HD)[:, :N].transpose(1, 0, 2).reshape(N, D)


def kernel(x, edge_attr, Wn, bn, Wee, bee, Wae, bae, ln_g, ln_b,
           in_Wa, in_We1, in_We2, in_Wc, in_bc,
           out_Wa, out_We1, out_We2, out_Wc, out_bc,
           EWself, EWmsg, Eb, Wfin, bfin, G1, g1b, G2, g2b,
           edge_index, batch):
    u = edge_index[0]
    v = edge_index[1]
    u2 = jnp.concatenate([u, u + N])
    zeros_half = jnp.zeros((N_PAD, HD), jnp.float32)

    gkey = jax.random.key(12345)
    gcols = []
    for i in range(2 * NUM_LAYERS):
        uu = jax.random.uniform(jax.random.fold_in(gkey, i), (N, 2),
                                minval=1e-6, maxval=1.0 - 1e-6)
        g = -jnp.log(-jnp.log(uu))
        bc = in_bc if i % 2 == 0 else out_bc
        gcols.append(((bc[0] + g[:, 0]).reshape(N, 1),
                      (bc[1] + g[:, 1]).reshape(N, 1)))
    wi0 = in_Wc[:, 0].reshape(D, 1)
    wi1 = in_Wc[:, 1].reshape(D, 1)
    wo0 = out_Wc[:, 0].reshape(D, 1)
    wo1 = out_Wc[:, 1].reshape(D, 1)

    env, p_in, p_out = _edge_prologue(edge_attr, Wee, bee, Wae, bae,
                                      in_We1, in_We2, out_We1, out_We2)
    env2 = _split_stack_edges(env)
    C_in = _unsplit_nodes(_sweep_linear()(v, _split_stack_edges(p_in), zeros_half))
    C_out = _unsplit_nodes(_sweep_linear()(v, _split_stack_edges(p_out), zeros_half))

    hL = _node_prologue(x, Wn, bn, ln_g, ln_b)

    for i in range(NUM_LAYERS):
        hWa_in, hWa_out = _premul(hL, in_Wa, out_Wa)
        S_in = _unsplit_nodes(_sweep_gather()(
            u2, v, _split_stack_nodes(hWa_in), zeros_half))
        S_out = _unsplit_nodes(_sweep_gather()(
            u2, v, _split_stack_nodes(hWa_out), zeros_half))
        gi0, gi1 = gcols[2 * i]
        go0, go1 = gcols[2 * i + 1]
        b_in, b_out, hB = _gates(S_in, S_out, C_in, C_out, hL,
                                 wi0, wi1, wo0, wo1, gi0, gi1, go0, go1)
        b2 = jnp.concatenate([b_out.reshape(N), b_out.reshape(N)])
        T = _unsplit_nodes(_sweep_msg()(
            u2, v, _split_stack_nodes(hB), b2, env2, zeros_half))
        hL = _update(hL, T, b_in, EWself[i], EWmsg[i], Eb[i], ln_g, ln_b)

    hF, gate = _final(hL, Wfin, bfin, G1, g1b, G2, g2b)
    oh = (batch.reshape(N, 1) == jnp.arange(NG, dtype=batch.dtype).reshape(1, NG)
          ).astype(jnp.float32)
    return _pool(gate, hF, oh)


# speedup vs baseline: 1.5249x; 1.0952x over previous
import functools

import jax
import jax.numpy as jnp
from jax import lax
from jax.experimental import pallas as pl
from jax.experimental.pallas import tpu as pltpu
from jax.experimental.pallas import tpu_sc as plsc

N = 50000
E = 800000
D = 64
HD = 32
NG = 16
NUM_LAYERS = 3

NTILES = 16
N_PAD = 50176
NPT = N_PAD // NTILES
EPT = E // NTILES
CH = 80
NCHUNK = EPT // CH

BN = 2000
NBN = N // BN
BE = 8000
NBE = E // BE


def _ln_rows(t, g, b):
    mu = jnp.mean(t, axis=-1, keepdims=True)
    var = jnp.mean((t - mu) * (t - mu), axis=-1, keepdims=True)
    return (t - mu) * lax.rsqrt(var + 1e-5) * g + b


def _prologue_body(x_ref, wn_ref, bn_ref, lng_ref, lnb_ref, hl_ref):
    h = jnp.dot(x_ref[...], wn_ref[...], preferred_element_type=jnp.float32)
    h = jnp.maximum(h + bn_ref[...], 0.0)
    hl_ref[...] = _ln_rows(h, lng_ref[...], lnb_ref[...])


def _node_prologue(x, Wn, bn, ln_g, ln_b):
    return pl.pallas_call(
        _prologue_body,
        grid=(NBN,),
        in_specs=[
            pl.BlockSpec((BN, 153), lambda i: (i, 0)),
            pl.BlockSpec((153, D), lambda i: (0, 0)),
            pl.BlockSpec((1, D), lambda i: (0, 0)),
            pl.BlockSpec((1, D), lambda i: (0, 0)),
            pl.BlockSpec((1, D), lambda i: (0, 0)),
        ],
        out_specs=pl.BlockSpec((BN, D), lambda i: (i, 0)),
        out_shape=jax.ShapeDtypeStruct((N, D), jnp.float32),
    )(x, Wn, bn.reshape(1, D), ln_g.reshape(1, D), ln_b.reshape(1, D))


def _edge_prologue_body(ea_ref, wee_ref, bee_ref, wae_ref, bae_ref,
                        wi1_ref, wi2_ref, wo1_ref, wo2_ref,
                        env_ref, pin_ref, pout_ref):
    ea = ea_ref[...]
    env = jnp.maximum(jnp.dot(ea, wee_ref[...], preferred_element_type=jnp.float32) + bee_ref[...], 0.0)
    act = jnp.maximum(jnp.dot(ea, wae_ref[...], preferred_element_type=jnp.float32) + bae_ref[...], 0.0)
    env_ref[...] = env
    pin_ref[...] = (jnp.dot(env, wi1_ref[...], preferred_element_type=jnp.float32)
                    + jnp.dot(act, wi2_ref[...], preferred_element_type=jnp.float32))
    pout_ref[...] = (jnp.dot(env, wo1_ref[...], preferred_element_type=jnp.float32)
                     + jnp.dot(act, wo2_ref[...], preferred_element_type=jnp.float32))


def _edge_prologue(edge_attr, Wee, bee, Wae, bae, iW1, iW2, oW1, oW2):
    wspec = pl.BlockSpec((7, D), lambda i: (0, 0))
    dspec = pl.BlockSpec((D, D), lambda i: (0, 0))
    bspec = pl.BlockSpec((1, D), lambda i: (0, 0))
    espec = pl.BlockSpec((BE, D), lambda i: (i, 0))
    return pl.pallas_call(
        _edge_prologue_body,
        grid=(NBE,),
        in_specs=[pl.BlockSpec((BE, 7), lambda i: (i, 0)),
                  wspec, bspec, wspec, bspec, dspec, dspec, dspec, dspec],
        out_specs=[espec, espec, espec],
        out_shape=[jax.ShapeDtypeStruct((E, D), jnp.float32)] * 3,
    )(edge_attr, Wee, bee.reshape(1, D), Wae, bae.reshape(1, D), iW1, iW2, oW1, oW2)


def _premul_body(hl_ref, wia_ref, woa_ref, hin_ref, hout_ref):
    hl = hl_ref[...]
    hin_ref[...] = jnp.dot(hl, wia_ref[...], preferred_element_type=jnp.float32)
    hout_ref[...] = jnp.dot(hl, woa_ref[...], preferred_element_type=jnp.float32)


def _premul(hL, in_Wa, out_Wa):
    nspec = pl.BlockSpec((BN, D), lambda i: (i, 0))
    dspec = pl.BlockSpec((D, D), lambda i: (0, 0))
    return pl.pallas_call(
        _premul_body,
        grid=(NBN,),
        in_specs=[nspec, dspec, dspec],
        out_specs=[nspec, nspec],
        out_shape=[jax.ShapeDtypeStruct((N, D), jnp.float32)] * 2,
    )(hL, in_Wa, out_Wa)


def _gates_body(sin_ref, sout_ref, cin_ref, cout_ref, hl_ref,
                wi0_ref, wi1_ref, wo0_ref, wo1_ref,
                gi0_ref, gi1_ref, go0_ref, go1_ref,
                bin_ref, bout_ref, hb_ref):
    a_in = jnp.maximum(sin_ref[...] + cin_ref[...], 0.0)
    a_out = jnp.maximum(sout_ref[...] + cout_ref[...], 0.0)
    li0 = jnp.dot(a_in, wi0_ref[...], preferred_element_type=jnp.float32) + gi0_ref[...]
    li1 = jnp.dot(a_in, wi1_ref[...], preferred_element_type=jnp.float32) + gi1_ref[...]
    lo0 = jnp.dot(a_out, wo0_ref[...], preferred_element_type=jnp.float32) + go0_ref[...]
    lo1 = jnp.dot(a_out, wo1_ref[...], preferred_element_type=jnp.float32) + go1_ref[...]
    b_in = (li0 >= li1).astype(jnp.float32)
    b_out = (lo0 >= lo1).astype(jnp.float32)
    bin_ref[...] = b_in
    bout_ref[...] = b_out
    hb_ref[...] = b_out * hl_ref[...]


def _gates(S_in, S_out, C_in, C_out, hL, wi0, wi1, wo0, wo1,
           gi0, gi1, go0, go1):
    nspec = pl.BlockSpec((BN, D), lambda i: (i, 0))
    cspec = pl.BlockSpec((D, 1), lambda i: (0, 0))
    vspec = pl.BlockSpec((BN, 1), lambda i: (i, 0))
    return pl.pallas_call(
        _gates_body,
        grid=(NBN,),
        in_specs=[nspec, nspec, nspec, nspec, nspec,
                  cspec, cspec, cspec, cspec,
                  vspec, vspec, vspec, vspec],
        out_specs=[vspec, vspec, nspec],
        out_shape=[jax.ShapeDtypeStruct((N, 1), jnp.float32),
                   jax.ShapeDtypeStruct((N, 1), jnp.float32),
                   jax.ShapeDtypeStruct((N, D), jnp.float32)],
    )(S_in, S_out, C_in, C_out, hL, wi0, wi1, wo0, wo1, gi0, gi1, go0, go1)


def _update_body(hl_ref, t_ref, bin_ref, ws_ref, wm_ref, eb_ref,
                 lng_ref, lnb_ref, out_ref):
    hl = hl_ref[...]
    agg = bin_ref[...] * t_ref[...]
    o = (jnp.dot(hl, ws_ref[...], preferred_element_type=jnp.float32)
         + jnp.dot(agg, wm_ref[...], preferred_element_type=jnp.float32)
         + eb_ref[...])
    r = hl + jnp.maximum(o, 0.0)
    out_ref[...] = _ln_rows(r, lng_ref[...], lnb_ref[...])


def _update(hL, T, b_in, Ws, Wm, Ebi, ln_g, ln_b):
    nspec = pl.BlockSpec((BN, D), lambda i: (i, 0))
    dspec = pl.BlockSpec((D, D), lambda i: (0, 0))
    bspec = pl.BlockSpec((1, D), lambda i: (0, 0))
    return pl.pallas_call(
        _update_body,
        grid=(NBN,),
        in_specs=[nspec, nspec, pl.BlockSpec((BN, 1), lambda i: (i, 0)),
                  dspec, dspec, bspec, bspec, bspec],
        out_specs=nspec,
        out_shape=jax.ShapeDtypeStruct((N, D), jnp.float32),
    )(hL, T, b_in, Ws, Wm, Ebi.reshape(1, D), ln_g.reshape(1, D), ln_b.reshape(1, D))


def _final_body(hl_ref, wf_ref, bf_ref, g1_ref, g1b_ref, g2_ref, g2b_ref,
                hf_ref, gate_ref):
    hf = jnp.dot(hl_ref[...], wf_ref[...], preferred_element_type=jnp.float32) + bf_ref[...]
    hf_ref[...] = hf
    t = jnp.maximum(jnp.dot(hf, g1_ref[...], preferred_element_type=jnp.float32) + g1b_ref[...], 0.0)
    gate_ref[...] = jnp.dot(t, g2_ref[...], preferred_element_type=jnp.float32) + g2b_ref[...]


def _final(hL, Wfin, bfin, G1, g1b, G2, g2b):
    nspec = pl.BlockSpec((BN, D), lambda i: (i, 0))
    dspec = pl.BlockSpec((D, D), lambda i: (0, 0))
    bspec = pl.BlockSpec((1, D), lambda i: (0, 0))
    return pl.pallas_call(
        _final_body,
        grid=(NBN,),
        in_specs=[nspec, dspec, bspec, dspec, bspec,
                  pl.BlockSpec((D, 1), lambda i: (0, 0)),
                  pl.BlockSpec((1, 1), lambda i: (0, 0))],
        out_specs=[nspec, pl.BlockSpec((BN, 1), lambda i: (i, 0))],
        out_shape=[jax.ShapeDtypeStruct((N, D), jnp.float32),
                   jax.ShapeDtypeStruct((N, 1), jnp.float32)],
    )(hL, Wfin, bfin.reshape(1, D), G1, g1b.reshape(1, D), G2,
      g2b.reshape(1, 1))


def _pool_body(gate_ref, hf_ref, oh_ref, out_ref, smax, sden, snum):
    p = pl.program_id(0)
    j = pl.program_id(1)
    oh = oh_ref[...]
    gate = gate_ref[...]

    @pl.when(jnp.logical_and(p == 0, j == 0))
    def _():
        smax[...] = jnp.full((8, NG), -jnp.inf, jnp.float32)

    @pl.when(p == 0)
    def _():
        m = jnp.max(jnp.where(oh > 0.0, gate, -jnp.inf), axis=0, keepdims=True)
        smax[0:1, :] = jnp.maximum(smax[0:1, :], m)

    @pl.when(jnp.logical_and(p == 1, j == 0))
    def _():
        sden[...] = jnp.zeros((8, NG), jnp.float32)
        snum[...] = jnp.zeros((NG, D), jnp.float32)

    @pl.when(p == 1)
    def _():
        gmax_node = jnp.sum(oh * smax[0:1, :], axis=1, keepdims=True)
        ex = jnp.exp(gate - gmax_node)
        sden[0:1, :] = sden[0:1, :] + jnp.sum(oh * ex, axis=0, keepdims=True)
        snum[...] = snum[...] + lax.dot_general(
            oh, ex * hf_ref[...], (((0,), (0,)), ((), ())),
            preferred_element_type=jnp.float32)

    @pl.when(jnp.logical_and(p == 1, j == NBN - 1))
    def _():
        den_col = jnp.reshape(sden[0:1, :], (NG, 1))
        out_ref[...] = snum[...] / (den_col + 1e-16)


def _pool(gate, hF, oh):
    return pl.pallas_call(
        _pool_body,
        grid=(2, NBN),
        in_specs=[pl.BlockSpec((BN, 1), lambda p, j: (j, 0)),
                  pl.BlockSpec((BN, D), lambda p, j: (j, 0)),
                  pl.BlockSpec((BN, NG), lambda p, j: (j, 0))],
        out_specs=pl.BlockSpec((NG, D), lambda p, j: (0, 0)),
        out_shape=jax.ShapeDtypeStruct((NG, D), jnp.float32),
        scratch_shapes=[pltpu.VMEM((8, NG), jnp.float32),
                        pltpu.VMEM((8, NG), jnp.float32),
                        pltpu.VMEM((NG, D), jnp.float32)],
    )(gate, hF, oh)


_MESH = plsc.VectorSubcoreMesh(core_axis_name="c", subcore_axis_name="s")
_SC_PARAMS = pltpu.CompilerParams(use_tc_tiling_on_sc=False,
                                  needs_layout_passes=False)


def _sweep_linear_kernel(v_hbm, rows2_hbm, zeros_hbm, out_hbm,
                         vidx_v, rows_v, shared):
    c = lax.axis_index("c")
    s = lax.axis_index("s")
    pltpu.sync_copy(zeros_hbm.at[pl.ds(s * NPT, NPT)],
                    shared.at[pl.ds(s * NPT, NPT)])
    plsc.subcore_barrier()

    def chunk(i, carry):
        base = s * EPT + i * CH
        pltpu.sync_copy(v_hbm.at[pl.ds(base, CH)], vidx_v)
        pltpu.sync_copy(rows2_hbm.at[pl.ds(c * E + base, CH)], rows_v)
        pltpu.sync_copy(rows_v, shared.at[vidx_v], add=True)
        return carry

    lax.fori_loop(0, NCHUNK, chunk, 0)
    plsc.subcore_barrier()
    pltpu.sync_copy(shared.at[pl.ds(s * NPT, NPT)],
                    out_hbm.at[pl.ds(c * N_PAD + s * NPT, NPT)])


def _sweep_gather_kernel(u2_hbm, v_hbm, table_hbm, zeros_hbm, out_hbm,
                         uidx0_v, uidx1_v, vidx0_v, vidx1_v,
                         rows0_v, rows1_v, sem0, sem1, shared):
    c = lax.axis_index("c")
    s = lax.axis_index("s")
    pltpu.sync_copy(zeros_hbm.at[pl.ds(s * NPT, NPT)],
                    shared.at[pl.ds(s * NPT, NPT)])
    plsc.subcore_barrier()
    ebase = s * EPT

    def fire(i, uidx_v, vidx_v, rows_v, sem):
        pltpu.sync_copy(u2_hbm.at[pl.ds(c * E + ebase + i * CH, CH)], uidx_v)
        pltpu.sync_copy(v_hbm.at[pl.ds(ebase + i * CH, CH)], vidx_v)
        pltpu.async_copy(table_hbm.at[uidx_v], rows_v, sem)

    def drain(uidx_v, vidx_v, rows_v, sem):
        pltpu.make_async_copy(table_hbm.at[uidx_v], rows_v, sem).wait()
        pltpu.sync_copy(rows_v, shared.at[vidx_v], add=True)

    fire(0, uidx0_v, vidx0_v, rows0_v, sem0)

    def pair(i2, carry):
        a = 2 * i2
        fire(a + 1, uidx1_v, vidx1_v, rows1_v, sem1)
        drain(uidx0_v, vidx0_v, rows0_v, sem0)
        fire(a + 2, uidx0_v, vidx0_v, rows0_v, sem0)
        drain(uidx1_v, vidx1_v, rows1_v, sem1)
        return carry

    lax.fori_loop(0, (NCHUNK - 1) // 2, pair, 0)
    drain(uidx0_v, vidx0_v, rows0_v, sem0)
    plsc.subcore_barrier()
    pltpu.sync_copy(shared.at[pl.ds(s * NPT, NPT)],
                    out_hbm.at[pl.ds(c * N_PAD + s * NPT, NPT)])


def _sweep_msg_kernel(u2_hbm, v_hbm, table_hbm, b2_hbm, env2_hbm,
                      zeros_hbm, out_hbm,
                      uidx_v, vidx_v, vidx2_v, rows_v, env_v, bval_v,
                      sem, semb,
                      uidx1_v, vidx1_v, vidx21_v, rows1_v, env1_v, bval1_v,
                      sem1, semb1, shared):
    c = lax.axis_index("c")
    s = lax.axis_index("s")
    pltpu.sync_copy(zeros_hbm.at[pl.ds(s * NPT, NPT)],
                    shared.at[pl.ds(s * NPT, NPT)])
    plsc.subcore_barrier()
    trash = jnp.full((16,), N, jnp.int32)
    ebase = s * EPT

    def fire(i, bufs):
        uidx_v, vidx_v, _, rows_v, env_v, bval_v, sem, semb = bufs
        pltpu.sync_copy(u2_hbm.at[pl.ds(c * E + ebase + i * CH, CH)], uidx_v)
        pltpu.sync_copy(v_hbm.at[pl.ds(ebase + i * CH, CH)], vidx_v)
        pltpu.sync_copy(env2_hbm.at[pl.ds(c * E + ebase + i * CH, CH)], env_v)
        pltpu.async_copy(table_hbm.at[uidx_v], rows_v, sem)
        pltpu.async_copy(b2_hbm.at[uidx_v], bval_v, semb)

    def drain(bufs):
        uidx_v, vidx_v, vidx2_v, rows_v, env_v, bval_v, sem, semb = bufs
        pltpu.make_async_copy(table_hbm.at[uidx_v], rows_v, sem).wait()
        pltpu.make_async_copy(b2_hbm.at[uidx_v], bval_v, semb).wait()
        for k in range(CH // 16):
            b = bval_v[pl.ds(16 * k, 16)]
            vi = vidx_v[pl.ds(16 * k, 16)]
            vidx2_v[pl.ds(16 * k, 16)] = jnp.where(b >= 0.5, vi, trash)
        pltpu.sync_copy(rows_v, shared.at[vidx_v], add=True)
        pltpu.sync_copy(env_v, shared.at[vidx2_v], add=True)

    bufs0 = (uidx_v, vidx_v, vidx2_v, rows_v, env_v, bval_v, sem, semb)
    bufs1 = (uidx1_v, vidx1_v, vidx21_v, rows1_v, env1_v, bval1_v, sem1, semb1)
    fire(0, bufs0)

    def pair(i2, carry):
        a = 2 * i2
        fire(a + 1, bufs1)
        drain(bufs0)
        fire(a + 2, bufs0)
        drain(bufs1)
        return carry

    lax.fori_loop(0, (NCHUNK - 1) // 2, pair, 0)
    drain(bufs0)
    plsc.subcore_barrier()
    pltpu.sync_copy(shared.at[pl.ds(s * NPT, NPT)],
                    out_hbm.at[pl.ds(c * N_PAD + s * NPT, NPT)])


_sweep_linear = functools.partial(
    pl.kernel, _sweep_linear_kernel, mesh=_MESH, compiler_params=_SC_PARAMS,
    out_type=jax.ShapeDtypeStruct((2 * N_PAD, HD), jnp.float32),
    scratch_types=[pltpu.VMEM((CH,), jnp.int32),
                   pltpu.VMEM((CH, HD), jnp.float32),
                   pltpu.VMEM_SHARED((N_PAD, HD), jnp.float32)],
)

_sweep_gather = functools.partial(
    pl.kernel, _sweep_gather_kernel, mesh=_MESH, compiler_params=_SC_PARAMS,
    out_type=jax.ShapeDtypeStruct((2 * N_PAD, HD), jnp.float32),
    scratch_types=[pltpu.VMEM((CH,), jnp.int32),
                   pltpu.VMEM((CH,), jnp.int32),
                   pltpu.VMEM((CH,), jnp.int32),
                   pltpu.VMEM((CH,), jnp.int32),
                   pltpu.VMEM((CH, HD), jnp.float32),
                   pltpu.VMEM((CH, HD), jnp.float32),
                   pltpu.SemaphoreType.DMA,
                   pltpu.SemaphoreType.DMA,
                   pltpu.VMEM_SHARED((N_PAD, HD), jnp.float32)],
)

_sweep_msg = functools.partial(
    pl.kernel, _sweep_msg_kernel, mesh=_MESH, compiler_params=_SC_PARAMS,
    out_type=jax.ShapeDtypeStruct((2 * N_PAD, HD), jnp.float32),
    scratch_types=[pltpu.VMEM((CH,), jnp.int32),
                   pltpu.VMEM((CH,), jnp.int32),
                   pltpu.VMEM((CH,), jnp.int32),
                   pltpu.VMEM((CH, HD), jnp.float32),
                   pltpu.VMEM((CH, HD), jnp.float32),
                   pltpu.VMEM((CH,), jnp.float32),
                   pltpu.SemaphoreType.DMA,
                   pltpu.SemaphoreType.DMA,
                   pltpu.VMEM((CH,), jnp.int32),
                   pltpu.VMEM((CH,), jnp.int32),
                   pltpu.VMEM((CH,), jnp.int32),
                   pltpu.VMEM((CH, HD), jnp.float32),
                   pltpu.VMEM((CH, HD), jnp.float32),
                   pltpu.VMEM((CH,), jnp.float32),
                   pltpu.SemaphoreType.DMA,
                   pltpu.SemaphoreType.DMA,
                   pltpu.VMEM_SHARED((N_PAD, HD), jnp.float32)],
)


def _split_stack_nodes(a):
    return a.reshape(N, 2, HD).transpose(1, 0, 2).reshape(2 * N, HD)


def _split_stack_edges(a):
    return a.reshape(E, 2, HD).transpose(1, 0, 2).reshape(2 * E, HD)


def _unsplit_nodes(a):
    return a.reshape(2, N_PAD, HD)[:, :N].transpose(1, 0, 2).reshape(N, D)


def kernel(x, edge_attr, Wn, bn, Wee, bee, Wae, bae, ln_g, ln_b,
           in_Wa, in_We1, in_We2, in_Wc, in_bc,
           out_Wa, out_We1, out_We2, out_Wc, out_bc,
           EWself, EWmsg, Eb, Wfin, bfin, G1, g1b, G2, g2b,
           edge_index, batch):
    u = edge_index[0]
    v = edge_index[1]
    u2 = jnp.concatenate([u, u + N])
    zeros_half = jnp.zeros((N_PAD, HD), jnp.float32)

    gkey = jax.random.key(12345)
    gcols = []
    for i in range(2 * NUM_LAYERS):
        uu = jax.random.uniform(jax.random.fold_in(gkey, i), (N, 2),
                                minval=1e-6, maxval=1.0 - 1e-6)
        g = -jnp.log(-jnp.log(uu))
        bc = in_bc if i % 2 == 0 else out_bc
        gcols.append(((bc[0] + g[:, 0]).reshape(N, 1),
                      (bc[1] + g[:, 1]).reshape(N, 1)))
    wi0 = in_Wc[:, 0].reshape(D, 1)
    wi1 = in_Wc[:, 1].reshape(D, 1)
    wo0 = out_Wc[:, 0].reshape(D, 1)
    wo1 = out_Wc[:, 1].reshape(D, 1)

    env, p_in, p_out = _edge_prologue(edge_attr, Wee, bee, Wae, bae,
                                      in_We1, in_We2, out_We1, out_We2)
    env2 = _split_stack_edges(env)
    C_in = _unsplit_nodes(_sweep_linear()(v, _split_stack_edges(p_in), zeros_half))
    C_out = _unsplit_nodes(_sweep_linear()(v, _split_stack_edges(p_out), zeros_half))

    hL = _node_prologue(x, Wn, bn, ln_g, ln_b)

    for i in range(NUM_LAYERS):
        hWa_in, hWa_out = _premul(hL, in_Wa, out_Wa)
        S_in = _unsplit_nodes(_sweep_gather()(
            u2, v, _split_stack_nodes(hWa_in), zeros_half))
        S_out = _unsplit_nodes(_sweep_gather()(
            u2, v, _split_stack_nodes(hWa_out), zeros_half))
        gi0, gi1 = gcols[2 * i]
        go0, go1 = gcols[2 * i + 1]
        b_in, b_out, hB = _gates(S_in, S_out, C_in, C_out, hL,
                                 wi0, wi1, wo0, wo1, gi0, gi1, go0, go1)
        b2 = jnp.concatenate([b_out.reshape(N), b_out.reshape(N)])
        T = _unsplit_nodes(_sweep_msg()(
            u2, v, _split_stack_nodes(hB), b2, env2, zeros_half))
        hL = _update(hL, T, b_in, EWself[i], EWmsg[i], Eb[i], ln_g, ln_b)

    hF, gate = _final(hL, Wfin, bfin, G1, g1b, G2, g2b)
    oh = (batch.reshape(N, 1) == jnp.arange(NG, dtype=batch.dtype).reshape(1, NG)
          ).astype(jnp.float32)
    return _pool(gate, hF, oh)
